# baseline (speedup 1.0000x reference)
import functools

import jax
import jax.numpy as jnp
from jax import lax
from jax.experimental import pallas as pl
from jax.experimental.pallas import tpu as pltpu
from jax.experimental.pallas import tpu_sc as plsc

NUM_REL = 1000000
K = 16
BATCH = 16384

_NUM_CORES = 2
_NUM_SUBCORES = 16
_NW = _NUM_CORES * _NUM_SUBCORES
_B_PER_W = BATCH // _NW
_G = _B_PER_W // 16
_LINE = 128

_mesh = plsc.VectorSubcoreMesh(core_axis_name="c", subcore_axis_name="s")


@functools.partial(
    pl.kernel,
    mesh=_mesh,
    out_type=jax.ShapeDtypeStruct((K, BATCH), jnp.float32),
    scratch_types=[
        pltpu.VMEM((_B_PER_W,), jnp.int32),
        pltpu.VMEM((3, 16, K, _LINE), jnp.float32),
        pltpu.VMEM((K, _B_PER_W), jnp.float32),
        pltpu.SemaphoreType.DMA,
    ],
    compiler_params=pltpu.CompilerParams(
        needs_layout_passes=False,
        skip_device_barrier=True,
        disable_bounds_checks=True,
        disable_semaphore_checks=True,
    ),
)
def _gather(idx_hbm, table_hbm, out_hbm, idx_v, blk_v, stage_v, sem):
    wid = lax.axis_index("s") * _NUM_CORES + lax.axis_index("c")
    base = wid * _B_PER_W
    iota = lax.iota(jnp.int32, 16)
    pltpu.sync_copy(idx_hbm.at[pl.ds(base, _B_PER_W)], idx_v)

    def copies(g, ring):
        v16 = idx_v[pl.ds(g * 16, 16)]
        out = []
        for j in range(16):
            col_al = pl.multiple_of(((v16[j] >> 7) << 7), _LINE)
            out.append(
                pltpu.make_async_copy(
                    table_hbm.at[:, pl.ds(col_al, _LINE)],
                    blk_v.at[ring, j],
                    sem,
                )
            )
        return v16, out

    def issue_group(g, ring):
        _, cps = copies(g, ring)
        for cp in cps:
            cp.start()

    def extract_group(g, ring):
        v16, cps = copies(g, ring)
        for cp in cps:
            cp.wait()
        for j in range(16):
            rem = jnp.full((16,), v16[j] & (_LINE - 1), jnp.int32)
            vals = plsc.load_gather(blk_v.at[ring, j], [iota, rem])
            plsc.store_scatter(
                stage_v, [iota, jnp.full((16,), g * 16 + j, jnp.int32)], vals
            )

    issue_group(0, 0)
    issue_group(1, 1)

    def body(g, carry):
        issue_group(g + 2, (g + 2) % 3)
        extract_group(g, g % 3)
        return carry

    lax.fori_loop(0, _G - 2, body, 0)
    extract_group(_G - 2, (_G - 2) % 3)
    extract_group(_G - 1, (_G - 1) % 3)

    pltpu.sync_copy(stage_v, out_hbm.at[:, pl.ds(base, _B_PER_W)])


def kernel(batch_relation, rel_attention):
    out_phys = _gather(batch_relation.astype(jnp.int32), rel_attention.T)
    return out_phys.T


# speedup vs baseline: 1.0376x; 1.0376x over previous
import functools

import jax
import jax.numpy as jnp
from jax import lax
from jax.experimental import pallas as pl
from jax.experimental.pallas import tpu as pltpu
from jax.experimental.pallas import tpu_sc as plsc

NUM_REL = 1000000
K = 16
BATCH = 16384

_NUM_CORES = 2
_NUM_SUBCORES = 16
_NW = _NUM_CORES * _NUM_SUBCORES
_B_PER_W = BATCH // _NW
_G = _B_PER_W // 16
_LINE = 128

_mesh = plsc.VectorSubcoreMesh(core_axis_name="c", subcore_axis_name="s")


@functools.partial(
    pl.kernel,
    mesh=_mesh,
    out_type=jax.ShapeDtypeStruct((K, BATCH), jnp.float32),
    scratch_types=[
        pltpu.VMEM((_B_PER_W,), jnp.int32),
        pltpu.VMEM((2, 16, K, _LINE), jnp.float32),
        pltpu.VMEM((K, _B_PER_W), jnp.float32),
        pltpu.SemaphoreType.DMA,
    ],
    compiler_params=pltpu.CompilerParams(needs_layout_passes=False),
)
def _gather(idx_hbm, table_hbm, out_hbm, idx_v, blk_v, stage_v, sem):
    wid = lax.axis_index("s") * _NUM_CORES + lax.axis_index("c")
    base = wid * _B_PER_W
    iota = lax.iota(jnp.int32, 16)
    pltpu.sync_copy(idx_hbm.at[pl.ds(base, _B_PER_W)], idx_v)

    def copies(g, ring):
        v16 = idx_v[pl.ds(g * 16, 16)]
        out = []
        for j in range(16):
            col_al = pl.multiple_of(((v16[j] >> 7) << 7), _LINE)
            out.append(
                pltpu.make_async_copy(
                    table_hbm.at[:, pl.ds(col_al, _LINE)],
                    blk_v.at[ring, j],
                    sem,
                )
            )
        return v16, out

    def issue_group(g, ring):
        _, cps = copies(g, ring)
        for cp in cps:
            cp.start()

    def extract_group(g, ring):
        v16, cps = copies(g, ring)
        for cp in cps:
            cp.wait()
        for j in range(16):
            rem = jnp.full((16,), v16[j] & (_LINE - 1), jnp.int32)
            vals = plsc.load_gather(blk_v.at[ring, j], [iota, rem])
            plsc.store_scatter(
                stage_v, [iota, jnp.full((16,), g * 16 + j, jnp.int32)], vals
            )

    issue_group(0, 0)

    def body(g, carry):
        issue_group(g + 1, (g + 1) & 1)
        extract_group(g, g & 1)
        return carry

    lax.fori_loop(0, _G - 1, body, 0)
    extract_group(_G - 1, (_G - 1) & 1)

    pltpu.sync_copy(stage_v, out_hbm.at[:, pl.ds(base, _B_PER_W)])


def kernel(batch_relation, rel_attention):
    out_phys = _gather(batch_relation.astype(jnp.int32), rel_attention.T)
    return out_phys.T
